# baseline (speedup 1.0000x reference)
import functools

import jax
import jax.numpy as jnp
from jax import lax
from jax.experimental import pallas as pl
from jax.experimental.pallas import tpu as pltpu
from jax.experimental.pallas import tpu_sc as plsc

N_NODES = 10000
N_PAD = 10016
NC = 2
NS = 16
NW = NC * NS
K = 128
DEG_W = 16
ROW_BLK = 2000


def _sc_mesh():
    return plsc.VectorSubcoreMesh(core_axis_name="c", subcore_axis_name="s")


def _sc_degree(dst3, ones, zeros):
    cpt = dst3.shape[1]

    @functools.partial(
        pl.kernel,
        mesh=_sc_mesh(),
        out_type=jax.ShapeDtypeStruct((NC, N_PAD, DEG_W), jnp.float32),
        compiler_params=pltpu.CompilerParams(use_tc_tiling_on_sc=False),
        scratch_types=[
            pltpu.VMEM((cpt, K), jnp.int32),
            pltpu.VMEM((K, DEG_W), jnp.float32),
            pltpu.VMEM_SHARED((N_PAD, DEG_W), jnp.float32),
        ],
    )
    def k(dst_hbm, ones_hbm, zeros_hbm, out_hbm, idx_v, ones_v, acc_sh):
        cid = lax.axis_index("c")
        sid = lax.axis_index("s")
        wid = cid * NS + sid
        pltpu.sync_copy(dst_hbm.at[wid], idx_v)
        pltpu.sync_copy(ones_hbm, ones_v)

        @pl.when(sid == 0)
        def _():
            pltpu.sync_copy(zeros_hbm, acc_sh)

        plsc.subcore_barrier()

        def body(c, carry):
            pltpu.sync_copy(ones_v, acc_sh.at[idx_v.at[c]], add=True)
            return carry

        lax.fori_loop(0, cpt, body, 0)

        plsc.subcore_barrier()

        @pl.when(sid == 0)
        def _():
            pltpu.sync_copy(acc_sh, out_hbm.at[cid])

    return k(dst3, ones, zeros)


def _sc_gather_scatter(table, src3, dst3, zeros, feat):
    cpt = src3.shape[1]

    @functools.partial(
        pl.kernel,
        mesh=_sc_mesh(),
        out_type=jax.ShapeDtypeStruct((NC, N_PAD, feat), jnp.float32),
        compiler_params=pltpu.CompilerParams(use_tc_tiling_on_sc=False),
        scratch_types=[
            pltpu.VMEM((cpt, K), jnp.int32),
            pltpu.VMEM((cpt, K), jnp.int32),
            pltpu.VMEM((K, feat), jnp.float32),
            pltpu.VMEM((K, feat), jnp.float32),
            pltpu.VMEM_SHARED((N_PAD, feat), jnp.float32),
            pltpu.SemaphoreType.DMA,
            pltpu.SemaphoreType.DMA,
        ],
    )
    def k(tab_hbm, src_hbm, dst_hbm, zeros_hbm, out_hbm,
          src_v, dst_v, b0, b1, acc_sh, sem0, sem1):
        cid = lax.axis_index("c")
        sid = lax.axis_index("s")
        wid = cid * NS + sid
        pltpu.sync_copy(src_hbm.at[wid], src_v)
        pltpu.sync_copy(dst_hbm.at[wid], dst_v)

        @pl.when(sid == 0)
        def _():
            pltpu.sync_copy(zeros_hbm, acc_sh)

        plsc.subcore_barrier()

        pltpu.async_copy(tab_hbm.at[src_v.at[0]], b0, sem0)

        def body(i, carry):
            t0 = 2 * i
            t1 = t0 + 1
            t2 = jnp.minimum(t1 + 1, cpt - 1)
            pltpu.make_async_copy(tab_hbm.at[src_v.at[t0]], b0, sem0).wait()
            pltpu.async_copy(tab_hbm.at[src_v.at[t1]], b1, sem1)
            pltpu.sync_copy(b0, acc_sh.at[dst_v.at[t0]], add=True)
            pltpu.make_async_copy(tab_hbm.at[src_v.at[t1]], b1, sem1).wait()
            pltpu.async_copy(tab_hbm.at[src_v.at[t2]], b0, sem0)
            pltpu.sync_copy(b1, acc_sh.at[dst_v.at[t1]], add=True)
            return carry

        lax.fori_loop(0, cpt // 2, body, 0)
        pltpu.make_async_copy(tab_hbm.at[src_v.at[cpt - 1]], b0, sem0).wait()
        if cpt % 2 == 1:
            pltpu.sync_copy(b0, acc_sh.at[dst_v.at[cpt - 1]], add=True)

        plsc.subcore_barrier()

        @pl.when(sid == 0)
        def _():
            pltpu.sync_copy(acc_sh, out_hbm.at[cid])

    return k(table, src3, dst3, zeros)


def _tc_first(x, W1, degp):
    n, in_ch = x.shape
    d_out = W1.shape[1]
    grid = (n // ROW_BLK,)

    def body(x_ref, w_ref, dp_ref, h_ref, g_ref, dis_ref):
        deg = dp_ref[0][:, 0:1] + dp_ref[1][:, 0:1] + 1.0
        dis = lax.rsqrt(deg)
        h = jnp.dot(x_ref[...], w_ref[...], preferred_element_type=jnp.float32)
        h_ref[...] = h
        g_ref[...] = dis * h
        dis_ref[...] = dis

    return pl.pallas_call(
        body,
        grid=grid,
        in_specs=[
            pl.BlockSpec((ROW_BLK, in_ch), lambda i: (i, 0)),
            pl.BlockSpec((in_ch, d_out), lambda i: (0, 0)),
            pl.BlockSpec((NC, ROW_BLK, DEG_W), lambda i: (0, i, 0)),
        ],
        out_specs=[
            pl.BlockSpec((ROW_BLK, d_out), lambda i: (i, 0)),
            pl.BlockSpec((ROW_BLK, d_out), lambda i: (i, 0)),
            pl.BlockSpec((ROW_BLK, 1), lambda i: (i, 0)),
        ],
        out_shape=[
            jax.ShapeDtypeStruct((n, d_out), jnp.float32),
            jax.ShapeDtypeStruct((n, d_out), jnp.float32),
            jax.ShapeDtypeStruct((n, 1), jnp.float32),
        ],
    )(x, W1, degp)


def _tc_mid(p, h1, dis, b1, W2):
    n, d1 = h1.shape
    d2 = W2.shape[1]
    grid = (n // ROW_BLK,)

    def body(p_ref, h_ref, dis_ref, b_ref, w_ref, h2_ref, g2_ref):
        dis = dis_ref[...]
        s = dis * (p_ref[0] + p_ref[1]) + (dis * dis) * h_ref[...] + b_ref[...]
        r = jnp.maximum(s, 0.0)
        h2 = jnp.dot(r, w_ref[...], preferred_element_type=jnp.float32)
        h2_ref[...] = h2
        g2_ref[...] = dis * h2

    return pl.pallas_call(
        body,
        grid=grid,
        in_specs=[
            pl.BlockSpec((NC, ROW_BLK, d1), lambda i: (0, i, 0)),
            pl.BlockSpec((ROW_BLK, d1), lambda i: (i, 0)),
            pl.BlockSpec((ROW_BLK, 1), lambda i: (i, 0)),
            pl.BlockSpec((1, d1), lambda i: (0, 0)),
            pl.BlockSpec((d1, d2), lambda i: (0, 0)),
        ],
        out_specs=[
            pl.BlockSpec((ROW_BLK, d2), lambda i: (i, 0)),
            pl.BlockSpec((ROW_BLK, d2), lambda i: (i, 0)),
        ],
        out_shape=[
            jax.ShapeDtypeStruct((n, d2), jnp.float32),
            jax.ShapeDtypeStruct((n, d2), jnp.float32),
        ],
    )(p, h1, dis, b1, W2)


def _tc_last(q, h2, dis, b2):
    n, d2 = h2.shape
    grid = (n // ROW_BLK,)

    def body(q_ref, h_ref, dis_ref, b_ref, o_ref):
        dis = dis_ref[...]
        o_ref[...] = (dis * (q_ref[0] + q_ref[1])
                      + (dis * dis) * h_ref[...] + b_ref[...])

    return pl.pallas_call(
        body,
        grid=grid,
        in_specs=[
            pl.BlockSpec((NC, ROW_BLK, d2), lambda i: (0, i, 0)),
            pl.BlockSpec((ROW_BLK, d2), lambda i: (i, 0)),
            pl.BlockSpec((ROW_BLK, 1), lambda i: (i, 0)),
            pl.BlockSpec((1, d2), lambda i: (0, 0)),
        ],
        out_specs=pl.BlockSpec((ROW_BLK, d2), lambda i: (i, 0)),
        out_shape=jax.ShapeDtypeStruct((n, d2), jnp.float32),
    )(q, h2, dis, b2)


def kernel(x, edge_index, W1, b1, W2, b2):
    n = x.shape[0]
    e = edge_index.shape[1]
    ei = edge_index.astype(jnp.int32)
    src, dst = ei[0], ei[1]

    epw = -(-e // NW)
    epw = -(-epw // 256) * 256
    e_pad = epw * NW
    pad = e_pad - e
    src_p = jnp.concatenate([src, jnp.zeros((pad,), jnp.int32)])
    trash = n + (jnp.arange(pad, dtype=jnp.int32) % (N_PAD - N_NODES))
    dst_p = jnp.concatenate([dst, trash])
    src3 = src_p.reshape(NW, epw // K, K)
    dst3 = dst_p.reshape(NW, epw // K, K)

    ones = jnp.ones((K, DEG_W), jnp.float32)
    degp = _sc_degree(dst3, ones, jnp.zeros((N_PAD, DEG_W), jnp.float32))

    h1, g1, dis = _tc_first(x, W1, degp)
    d1 = W1.shape[1]
    p1 = _sc_gather_scatter(g1, src3, dst3,
                            jnp.zeros((N_PAD, d1), jnp.float32), d1)
    h2, g2 = _tc_mid(p1, h1, dis, b1.reshape(1, -1), W2)
    d2 = W2.shape[1]
    q = _sc_gather_scatter(g2, src3, dst3,
                           jnp.zeros((N_PAD, d2), jnp.float32), d2)
    return _tc_last(q, h2, dis, b2.reshape(1, -1))


# speedup vs baseline: 1.4678x; 1.4678x over previous
import functools

import jax
import jax.numpy as jnp
from jax import lax
from jax.experimental import pallas as pl
from jax.experimental.pallas import tpu as pltpu
from jax.experimental.pallas import tpu_sc as plsc

N_NODES = 10000
N_PAD = 10016
NC = 2
NS = 16
NW = NC * NS
K = 128
DEG_W = 16
ROW_BLK = 2000


def _sc_mesh():
    return plsc.VectorSubcoreMesh(core_axis_name="c", subcore_axis_name="s")


def _sc_degree(dst3, ones, zeros):
    cpt = dst3.shape[1]

    @functools.partial(
        pl.kernel,
        mesh=_sc_mesh(),
        out_type=jax.ShapeDtypeStruct((NC, N_PAD, DEG_W), jnp.float32),
        compiler_params=pltpu.CompilerParams(use_tc_tiling_on_sc=False),
        scratch_types=[
            pltpu.VMEM((cpt, K), jnp.int32),
            pltpu.VMEM((K, DEG_W), jnp.float32),
            pltpu.VMEM_SHARED((N_PAD, DEG_W), jnp.float32),
        ],
    )
    def k(dst_hbm, ones_hbm, zeros_hbm, out_hbm, idx_v, ones_v, acc_sh):
        cid = lax.axis_index("c")
        sid = lax.axis_index("s")
        wid = cid * NS + sid
        pltpu.sync_copy(dst_hbm.at[wid], idx_v)
        pltpu.sync_copy(ones_hbm, ones_v)

        @pl.when(sid == 0)
        def _():
            pltpu.sync_copy(zeros_hbm, acc_sh)

        plsc.subcore_barrier()

        def body(c, carry):
            pltpu.sync_copy(ones_v, acc_sh.at[idx_v.at[c]], add=True)
            return carry

        lax.fori_loop(0, cpt, body, 0)

        plsc.subcore_barrier()

        @pl.when(sid == 0)
        def _():
            pltpu.sync_copy(acc_sh, out_hbm.at[cid])

    return k(dst3, ones, zeros)


def _sc_gather_scatter(table, src3, dst3, zeros, feat):
    cpt = src3.shape[1]

    @functools.partial(
        pl.kernel,
        mesh=_sc_mesh(),
        out_type=jax.ShapeDtypeStruct((NC, N_PAD, feat), jnp.float32),
        compiler_params=pltpu.CompilerParams(use_tc_tiling_on_sc=False),
        scratch_types=[
            pltpu.VMEM((cpt, K), jnp.int32),
            pltpu.VMEM((cpt, K), jnp.int32),
            pltpu.VMEM((K, feat), jnp.float32),
            pltpu.VMEM((K, feat), jnp.float32),
            pltpu.VMEM((K, feat), jnp.float32),
            pltpu.VMEM((K, feat), jnp.float32),
            pltpu.VMEM_SHARED((N_PAD, feat), jnp.float32),
            pltpu.SemaphoreType.DMA,
            pltpu.SemaphoreType.DMA,
            pltpu.SemaphoreType.DMA,
            pltpu.SemaphoreType.DMA,
            pltpu.SemaphoreType.DMA,
            pltpu.SemaphoreType.DMA,
            pltpu.SemaphoreType.DMA,
            pltpu.SemaphoreType.DMA,
        ],
    )
    def k(tab_hbm, src_hbm, dst_hbm, zeros_hbm, out_hbm,
          src_v, dst_v, b0, b1, b2, b3, acc_sh,
          gs0, gs1, gs2, gs3, ss0, ss1, ss2, ss3):
        cid = lax.axis_index("c")
        sid = lax.axis_index("s")
        wid = cid * NS + sid
        pltpu.sync_copy(src_hbm.at[wid], src_v)
        pltpu.sync_copy(dst_hbm.at[wid], dst_v)

        @pl.when(sid == 0)
        def _():
            pltpu.sync_copy(zeros_hbm, acc_sh)

        plsc.subcore_barrier()

        bufs = (b0, b1, b2, b3)
        gsem = (gs0, gs1, gs2, gs3)
        ssem = (ss0, ss1, ss2, ss3)
        assert cpt % 4 == 3

        def wait_scatter(x, t):
            pltpu.make_async_copy(bufs[x], acc_sh.at[dst_v.at[t]],
                                  ssem[x]).wait()

        def wait_gather(x, t):
            pltpu.make_async_copy(tab_hbm.at[src_v.at[t]], bufs[x],
                                  gsem[x]).wait()

        pltpu.async_copy(tab_hbm.at[src_v.at[0]], b0, gs0)
        pltpu.async_copy(tab_hbm.at[src_v.at[1]], b1, gs1)

        def body(i, carry):
            for j in range(4):
                t = 4 * i + j
                x = j
                xp = (j + 2) % 4
                if j < 2:
                    @pl.when(i > 0)
                    def _():
                        wait_scatter(xp, t - 2)
                else:
                    wait_scatter(xp, t - 2)
                u = jnp.minimum(t + 2, cpt - 1)
                pltpu.async_copy(tab_hbm.at[src_v.at[u]], bufs[xp], gsem[xp])
                wait_gather(x, t)
                pltpu.async_copy(bufs[x], acc_sh.at[dst_v.at[t]], ssem[x],
                                 add=True)
            return carry

        lax.fori_loop(0, cpt // 4, body, 0)

        t0 = cpt - 3
        for t in (t0, t0 + 1, t0 + 2):
            x = t % 4
            xp = (t + 2) % 4
            wait_scatter(xp, t - 2)
            if t + 2 <= cpt - 1:
                pltpu.async_copy(tab_hbm.at[src_v.at[t + 2]], bufs[xp],
                                 gsem[xp])
            wait_gather(x, t)
            pltpu.async_copy(bufs[x], acc_sh.at[dst_v.at[t]], ssem[x],
                             add=True)
        wait_scatter((t0 + 1) % 4, t0 + 1)
        wait_scatter((t0 + 2) % 4, t0 + 2)

        plsc.subcore_barrier()

        @pl.when(sid == 0)
        def _():
            pltpu.sync_copy(acc_sh, out_hbm.at[cid])

    return k(table, src3, dst3, zeros)


def _tc_first(x, W1, degp):
    n, in_ch = x.shape
    d_out = W1.shape[1]
    grid = (n // ROW_BLK,)

    def body(x_ref, w_ref, dp_ref, h_ref, g_ref, dis_ref):
        deg = dp_ref[0][:, 0:1] + dp_ref[1][:, 0:1] + 1.0
        dis = lax.rsqrt(deg)
        h = jnp.dot(x_ref[...], w_ref[...], preferred_element_type=jnp.float32)
        h_ref[...] = h
        g_ref[...] = dis * h
        dis_ref[...] = dis

    return pl.pallas_call(
        body,
        grid=grid,
        in_specs=[
            pl.BlockSpec((ROW_BLK, in_ch), lambda i: (i, 0)),
            pl.BlockSpec((in_ch, d_out), lambda i: (0, 0)),
            pl.BlockSpec((NC, ROW_BLK, DEG_W), lambda i: (0, i, 0)),
        ],
        out_specs=[
            pl.BlockSpec((ROW_BLK, d_out), lambda i: (i, 0)),
            pl.BlockSpec((ROW_BLK, d_out), lambda i: (i, 0)),
            pl.BlockSpec((ROW_BLK, 1), lambda i: (i, 0)),
        ],
        out_shape=[
            jax.ShapeDtypeStruct((n, d_out), jnp.float32),
            jax.ShapeDtypeStruct((n, d_out), jnp.float32),
            jax.ShapeDtypeStruct((n, 1), jnp.float32),
        ],
    )(x, W1, degp)


def _tc_mid(p, h1, dis, b1, W2):
    n, d1 = h1.shape
    d2 = W2.shape[1]
    grid = (n // ROW_BLK,)

    def body(p_ref, h_ref, dis_ref, b_ref, w_ref, h2_ref, g2_ref):
        dis = dis_ref[...]
        s = dis * (p_ref[0] + p_ref[1]) + (dis * dis) * h_ref[...] + b_ref[...]
        r = jnp.maximum(s, 0.0)
        h2 = jnp.dot(r, w_ref[...], preferred_element_type=jnp.float32)
        h2_ref[...] = h2
        g2_ref[...] = dis * h2

    return pl.pallas_call(
        body,
        grid=grid,
        in_specs=[
            pl.BlockSpec((NC, ROW_BLK, d1), lambda i: (0, i, 0)),
            pl.BlockSpec((ROW_BLK, d1), lambda i: (i, 0)),
            pl.BlockSpec((ROW_BLK, 1), lambda i: (i, 0)),
            pl.BlockSpec((1, d1), lambda i: (0, 0)),
            pl.BlockSpec((d1, d2), lambda i: (0, 0)),
        ],
        out_specs=[
            pl.BlockSpec((ROW_BLK, d2), lambda i: (i, 0)),
            pl.BlockSpec((ROW_BLK, d2), lambda i: (i, 0)),
        ],
        out_shape=[
            jax.ShapeDtypeStruct((n, d2), jnp.float32),
            jax.ShapeDtypeStruct((n, d2), jnp.float32),
        ],
    )(p, h1, dis, b1, W2)


def _tc_last(q, h2, dis, b2):
    n, d2 = h2.shape
    grid = (n // ROW_BLK,)

    def body(q_ref, h_ref, dis_ref, b_ref, o_ref):
        dis = dis_ref[...]
        o_ref[...] = (dis * (q_ref[0] + q_ref[1])
                      + (dis * dis) * h_ref[...] + b_ref[...])

    return pl.pallas_call(
        body,
        grid=grid,
        in_specs=[
            pl.BlockSpec((NC, ROW_BLK, d2), lambda i: (0, i, 0)),
            pl.BlockSpec((ROW_BLK, d2), lambda i: (i, 0)),
            pl.BlockSpec((ROW_BLK, 1), lambda i: (i, 0)),
            pl.BlockSpec((1, d2), lambda i: (0, 0)),
        ],
        out_specs=pl.BlockSpec((ROW_BLK, d2), lambda i: (i, 0)),
        out_shape=jax.ShapeDtypeStruct((n, d2), jnp.float32),
    )(q, h2, dis, b2)


def kernel(x, edge_index, W1, b1, W2, b2):
    n = x.shape[0]
    e = edge_index.shape[1]
    ei = edge_index.astype(jnp.int32)
    src, dst = ei[0], ei[1]

    epw = -(-e // NW)
    epw = -(-epw // K) * K
    e_pad = epw * NW
    pad = e_pad - e
    src_p = jnp.concatenate([src, jnp.zeros((pad,), jnp.int32)])
    trash = n + (jnp.arange(pad, dtype=jnp.int32) % (N_PAD - N_NODES))
    dst_p = jnp.concatenate([dst, trash])
    src3 = src_p.reshape(NW, epw // K, K)
    dst3 = dst_p.reshape(NW, epw // K, K)

    ones = jnp.ones((K, DEG_W), jnp.float32)
    degp = _sc_degree(dst3, ones, jnp.zeros((N_PAD, DEG_W), jnp.float32))

    h1, g1, dis = _tc_first(x, W1, degp)
    d1 = W1.shape[1]
    p1 = _sc_gather_scatter(g1, src3, dst3,
                            jnp.zeros((N_PAD, d1), jnp.float32), d1)
    h2, g2 = _tc_mid(p1, h1, dis, b1.reshape(1, -1), W2)
    d2 = W2.shape[1]
    q = _sc_gather_scatter(g2, src3, dst3,
                           jnp.zeros((N_PAD, d2), jnp.float32), d2)
    return _tc_last(q, h2, dis, b2.reshape(1, -1))
